# Initial kernel scaffold; baseline (speedup 1.0000x reference)
import jax
import jax.numpy as jnp
from jax.experimental import pallas as pl


def kernel(context, table_p, targets):
    raise NotImplementedError("write your pallas kernel here")


# speedup vs baseline: 1.1836x; 1.1836x over previous
import jax
import jax.numpy as jnp
from jax import lax
from jax.experimental import pallas as pl
from jax.experimental.pallas import tpu as pltpu

_TILE = 4096


def _round_up(x, m):
    return (x + m - 1) // m * m


def _row_lse_kernel(table_ref, lse_ref):
    t = table_ref[...]
    m = jnp.max(t, axis=1, keepdims=True)
    lse_ref[...] = m + jnp.log(jnp.sum(jnp.exp(t - m), axis=1, keepdims=True))


def _gather_loss_kernel(ids_ref, tgt_ref, table_ref, lse_ref, out_ref, part_ref):
    ids = ids_ref[...]
    tgt = tgt_ref[...]
    tn = ids.shape[0]
    vp = table_ref.shape[1]
    col = lax.broadcasted_iota(jnp.int32, (tn, vp), 1)
    sel = col == ids
    onehot = sel.astype(jnp.float32)
    logits = jnp.dot(onehot, table_ref[...],
                     preferred_element_type=jnp.float32)
    out_ref[...] = logits

    lse_b = lse_ref[0:1, :]
    contrib = jnp.where(sel, lse_b, 0.0) - jnp.where(col == tgt, logits, 0.0)
    s = jnp.sum(contrib, axis=1, keepdims=True)
    s = jnp.sum(s, axis=0, keepdims=True)
    part_ref[...] = jnp.broadcast_to(s[None], (1, 1, 128))


def kernel(context, table_p, targets):
    B, T = context.shape
    N = B * T
    Vp = table_p.shape[0]
    tile = min(_TILE, _round_up(N, 8))
    Np = _round_up(N, tile)
    grid_n = Np // tile

    ids = context.reshape(N).astype(jnp.int32)
    tgt = targets.reshape(N).astype(jnp.int32)
    if Np != N:
        ids = jnp.pad(ids, (0, Np - N), constant_values=-1)
        tgt = jnp.pad(tgt, (0, Np - N), constant_values=-1)
    ids = ids.reshape(Np, 1)
    tgt = tgt.reshape(Np, 1)

    lse = pl.pallas_call(
        _row_lse_kernel,
        out_shape=jax.ShapeDtypeStruct((Vp, 1), jnp.float32),
        compiler_params=pltpu.CompilerParams(),
    )(table_p)
    lse8 = jnp.broadcast_to(lse.reshape(1, Vp), (8, Vp))

    cp = pltpu.CompilerParams(
        dimension_semantics=("parallel",),
        vmem_limit_bytes=64 * 1024 * 1024,
    )
    logits_p, parts = pl.pallas_call(
        _gather_loss_kernel,
        out_shape=(
            jax.ShapeDtypeStruct((Np, Vp), jnp.float32),
            jax.ShapeDtypeStruct((grid_n, 1, 128), jnp.float32),
        ),
        grid=(grid_n,),
        in_specs=[
            pl.BlockSpec((tile, 1), lambda i: (i, 0)),
            pl.BlockSpec((tile, 1), lambda i: (i, 0)),
            pl.BlockSpec((Vp, Vp), lambda i: (0, 0)),
            pl.BlockSpec((8, Vp), lambda i: (0, 0)),
        ],
        out_specs=(
            pl.BlockSpec((tile, Vp), lambda i: (i, 0)),
            pl.BlockSpec((1, 1, 128), lambda i: (i, 0, 0)),
        ),
        compiler_params=cp,
    )(ids, tgt, table_p, lse8)

    loss = jnp.sum(parts[:, 0, 0]) / jnp.float32(N)
    logits = logits_p if Np == N else logits_p[:N]
    return logits, loss


# speedup vs baseline: 8.9252x; 7.5408x over previous
import jax
import jax.numpy as jnp
from jax import lax
from jax.experimental import pallas as pl
from jax.experimental.pallas import tpu as pltpu


def _loss_table_kernel(table_ref, d_ref):
    t = table_ref[...]
    m = jnp.max(t, axis=1, keepdims=True)
    lse = m + jnp.log(jnp.sum(jnp.exp(t - m), axis=1, keepdims=True))
    d_ref[...] = lse - t


def _gather_loss_kernel(ids_ref, tgt_ref, table_ref, d_ref, out_ref, part_ref):
    rr, tt = ids_ref.shape
    vp = table_ref.shape[0]
    iota = lax.broadcasted_iota(jnp.int32, (vp, tt), 0)
    a_pieces = []
    g_pieces = []
    for r in range(rr):
        a_pieces.append((iota == ids_ref[r:r + 1, :]).astype(jnp.float32))
        g_pieces.append((iota == tgt_ref[r:r + 1, :]).astype(jnp.float32))
    a = jnp.concatenate(a_pieces, axis=1)
    g = jnp.concatenate(g_pieces, axis=1)

    out_ref[...] = lax.dot_general(
        a, table_ref[...], (((0,), (0,)), ((), ())),
        preferred_element_type=jnp.float32)

    c = lax.dot_general(a, g, (((1,), (1,)), ((), ())),
                        preferred_element_type=jnp.float32)
    s = jnp.sum(c * d_ref[...], axis=1, keepdims=True)
    s = jnp.sum(s, axis=0, keepdims=True)
    part_ref[...] = jnp.broadcast_to(s[None], (1, 1, 128))


def kernel(context, table_p, targets):
    B, T = context.shape
    N = B * T
    Vp = table_p.shape[0]
    R = 8
    grid_n = B // R

    d_tab = pl.pallas_call(
        _loss_table_kernel,
        out_shape=jax.ShapeDtypeStruct((Vp, Vp), jnp.float32),
        compiler_params=pltpu.CompilerParams(),
    )(table_p)

    cp = pltpu.CompilerParams(
        dimension_semantics=("parallel",),
        vmem_limit_bytes=100 * 1024 * 1024,
    )
    logits, parts = pl.pallas_call(
        _gather_loss_kernel,
        out_shape=(
            jax.ShapeDtypeStruct((N, Vp), jnp.float32),
            jax.ShapeDtypeStruct((grid_n, 1, 128), jnp.float32),
        ),
        grid=(grid_n,),
        in_specs=[
            pl.BlockSpec((R, T), lambda i: (i, 0)),
            pl.BlockSpec((R, T), lambda i: (i, 0)),
            pl.BlockSpec((Vp, Vp), lambda i: (0, 0)),
            pl.BlockSpec((Vp, Vp), lambda i: (0, 0)),
        ],
        out_specs=(
            pl.BlockSpec((R * T, Vp), lambda i: (i, 0)),
            pl.BlockSpec((1, 1, 128), lambda i: (i, 0, 0)),
        ),
        compiler_params=cp,
    )(context.astype(jnp.int32), targets.astype(jnp.int32), table_p, d_tab)

    loss = jnp.sum(parts[:, 0, 0]) / jnp.float32(N)
    return logits, loss


# speedup vs baseline: 9.5758x; 1.0729x over previous
import jax
import jax.numpy as jnp
from jax import lax
from jax.experimental import pallas as pl
from jax.experimental.pallas import tpu as pltpu


def _loss_table_kernel(table_ref, d_ref):
    t = table_ref[...]
    m = jnp.max(t, axis=1, keepdims=True)
    lse = m + jnp.log(jnp.sum(jnp.exp(t - m), axis=1, keepdims=True))
    d_ref[...] = lse - t


def _gather_loss_kernel(ids_ref, tgt_ref, table_ref, d_ref, out_ref, part_ref):
    rr, tt = ids_ref.shape
    vp = table_ref.shape[0]
    iota = lax.broadcasted_iota(jnp.int32, (vp, tt), 0)
    half = rr // 2
    cs = []
    for h in range(2):
        a_pieces = []
        g_pieces = []
        for r in range(h * half, (h + 1) * half):
            a_pieces.append((iota == ids_ref[r:r + 1, :]).astype(jnp.float32))
            g_pieces.append((iota == tgt_ref[r:r + 1, :]).astype(jnp.float32))
        a = jnp.concatenate(a_pieces, axis=1)
        g = jnp.concatenate(g_pieces, axis=1)

        out_ref[h * half * tt:(h + 1) * half * tt, :] = lax.dot_general(
            a, table_ref[...], (((0,), (0,)), ((), ())),
            preferred_element_type=jnp.float32)

        cs.append(lax.dot_general(a, g, (((1,), (1,)), ((), ())),
                                  preferred_element_type=jnp.float32))
    s = jnp.sum((cs[0] + cs[1]) * d_ref[...], axis=1, keepdims=True)
    s = jnp.sum(s, axis=0, keepdims=True)
    part_ref[...] = jnp.broadcast_to(s[None], (1, 1, 128))


def kernel(context, table_p, targets):
    B, T = context.shape
    N = B * T
    Vp = table_p.shape[0]
    R = 16
    grid_n = B // R

    d_tab = pl.pallas_call(
        _loss_table_kernel,
        out_shape=jax.ShapeDtypeStruct((Vp, Vp), jnp.float32),
        compiler_params=pltpu.CompilerParams(),
    )(table_p)

    cp = pltpu.CompilerParams(
        dimension_semantics=("parallel",),
        vmem_limit_bytes=100 * 1024 * 1024,
    )
    logits, parts = pl.pallas_call(
        _gather_loss_kernel,
        out_shape=(
            jax.ShapeDtypeStruct((N, Vp), jnp.float32),
            jax.ShapeDtypeStruct((grid_n, 1, 128), jnp.float32),
        ),
        grid=(grid_n,),
        in_specs=[
            pl.BlockSpec((R, T), lambda i: (i, 0)),
            pl.BlockSpec((R, T), lambda i: (i, 0)),
            pl.BlockSpec((Vp, Vp), lambda i: (0, 0)),
            pl.BlockSpec((Vp, Vp), lambda i: (0, 0)),
        ],
        out_specs=(
            pl.BlockSpec((R * T, Vp), lambda i: (i, 0)),
            pl.BlockSpec((1, 1, 128), lambda i: (i, 0, 0)),
        ),
        compiler_params=cp,
    )(context.astype(jnp.int32), targets.astype(jnp.int32), table_p, d_tab)

    loss = jnp.sum(parts[:, 0, 0]) / jnp.float32(N)
    return logits, loss
